# baseline (speedup 1.0000x reference)
import functools

import jax
import jax.numpy as jnp
from jax import lax
from jax.experimental import pallas as pl
from jax.experimental.pallas import tpu as pltpu
from jax.experimental.pallas import tpu_sc as plsc

N_CAT = 26
EMB = 16
PADV = 100096
NROW = 100000
NLANE = N_CAT * EMB
SPLIT = 216
Q = 4096
NW = 32

_SC_PARAMS = dict(
    compiler_params=pltpu.CompilerParams(use_tc_tiling_on_sc=False,
                                         needs_layout_passes=False),
    mesh=plsc.VectorSubcoreMesh(core_axis_name="c", subcore_axis_name="s"),
)


def _sc_cidx(cat1d, nodes, batch, n_cat):
    nq = batch // Q

    @functools.partial(
        pl.kernel,
        out_type=jax.ShapeDtypeStruct((n_cat * batch,), jnp.int32),
        scratch_types=[
            pltpu.VMEM((PADV,), jnp.int32),
            pltpu.VMEM((Q,), jnp.int32),
            pltpu.VMEM((Q,), jnp.int32),
        ],
        **_SC_PARAMS,
    )
    def k(cat_hbm, nodes_hbm, cidx_hbm, big_vm, nodes_vm, vals_vm):
        wid = lax.axis_index("s") * 2 + lax.axis_index("c")

        @pl.when(wid < n_cat)
        def _():
            pltpu.sync_copy(cat_hbm.at[pl.ds(wid * NROW, NROW)],
                            big_vm.at[pl.ds(0, NROW)])
            for q in range(nq):
                pltpu.sync_copy(nodes_hbm.at[pl.ds(q * Q, Q)], nodes_vm)

                def g(i, carry):
                    for u in range(8):
                        i16 = nodes_vm[pl.ds(i * 128 + u * 16, 16)]
                        vals_vm[pl.ds(i * 128 + u * 16, 16)] = \
                            plsc.load_gather(big_vm, [i16])
                    return carry
                lax.fori_loop(0, Q // 128, g, 0)
                pltpu.sync_copy(vals_vm,
                                cidx_hbm.at[pl.ds(wid * batch + q * Q, Q)])

    return k(cat1d, nodes)


def _sc_values(t1d, cidx1d, batch, lane_lo, lane_hi, num1d=None, nodes=None,
               n_num=0):
    nq = batch // Q
    n_tab = lane_hi - lane_lo
    out_rows = n_tab + n_num
    extra = (num1d, nodes) if n_num else ()

    @functools.partial(
        pl.kernel,
        out_type=jax.ShapeDtypeStruct((out_rows * batch,), jnp.float32),
        scratch_types=[
            pltpu.VMEM((PADV,), jnp.float32),
            pltpu.VMEM((batch,), jnp.int32),
            pltpu.VMEM((Q,), jnp.float32),
        ],
        **_SC_PARAMS,
    )
    def k(t_hbm, cidx_hbm, *rest):
        (num_hbm, nodes_hbm) = rest[:2] if n_num else (None, None)
        out_hbm, big_vm, cidx_vm, vals_vm = rest[-4:]
        wid = lax.axis_index("s") * 2 + lax.axis_index("c")

        def quarters(out_row):
            for q in range(nq):
                def g(i, carry):
                    for u in range(8):
                        i16 = cidx_vm[pl.ds(q * Q + i * 128 + u * 16, 16)]
                        vals_vm[pl.ds(i * 128 + u * 16, 16)] = \
                            plsc.load_gather(big_vm, [i16])
                    return carry
                lax.fori_loop(0, Q // 128, g, 0)
                pltpu.sync_copy(
                    vals_vm,
                    out_hbm.at[pl.ds(out_row * batch + q * Q, Q)])

        def tjob(t, carry):
            j = wid + NW * t

            @pl.when(j < n_tab)
            def _():
                f = (lane_lo + j) // EMB
                pltpu.sync_copy(cidx_hbm.at[pl.ds(f * batch, batch)], cidx_vm)
                pltpu.sync_copy(t_hbm.at[pl.ds(j * PADV, PADV)], big_vm)
                quarters(j)
            return carry
        lax.fori_loop(0, (n_tab + NW - 1) // NW, tjob, 0)

        if n_num:
            kk = wid - (NW - n_num)

            @pl.when(kk >= 0)
            def _num_tail():
                pltpu.sync_copy(nodes_hbm, cidx_vm)
                pltpu.sync_copy(num_hbm.at[pl.ds(kk * NROW, PADV)], big_vm)
                quarters(n_tab + kk)

    return k(t1d, cidx1d, *extra)


def _tc_repack(t_t, blk_lo, n_blk):
    def body(in_ref, out_ref):
        for j in range(8):
            out_ref[pl.ds(j * PADV, PADV)] = in_ref[0, j, :]

    return pl.pallas_call(
        body,
        grid=(n_blk,),
        in_specs=[pl.BlockSpec(
            (1, 8, PADV), lambda b: ((b + blk_lo) // 2, (b + blk_lo) % 2, 0))],
        out_specs=pl.BlockSpec((8 * PADV,), lambda b: (b,)),
        out_shape=jax.ShapeDtypeStruct((n_blk * 8 * PADV,), jnp.float32),
    )(t_t)


def _tc_matmul(emb, w, b2d, batch, acc_in=None):
    blk = 2048
    kd = emb.shape[0]
    extra = (acc_in,) if acc_in is not None else ()

    def body(emb_ref, w_ref, b_ref, *rest):
        acc_ref = rest[0] if acc_in is not None else None
        out_ref = rest[-1]
        acc = jax.lax.dot_general(
            emb_ref[...], w_ref[...],
            dimension_numbers=(((0,), (0,)), ((), ())),
            preferred_element_type=jnp.float32,
            precision=lax.Precision.HIGHEST)
        acc += b_ref[0:1, :] if acc_in is None else acc_ref[...]
        out_ref[...] = acc

    return pl.pallas_call(
        body,
        grid=(batch // blk,),
        in_specs=[
            pl.BlockSpec((kd, blk), lambda i: (0, i)),
            pl.BlockSpec((kd, 64), lambda i: (0, 0)),
            pl.BlockSpec((8, 64), lambda i: (0, 0)),
        ] + ([pl.BlockSpec((blk, 64), lambda i: (i, 0))] if extra else []),
        out_specs=pl.BlockSpec((blk, 64), lambda i: (i, 0)),
        out_shape=jax.ShapeDtypeStruct((batch, 64), jnp.float32),
    )(emb, w, b2d, *extra)


def kernel(tables, num_feats, W, b, cat_idx, nodes_v):
    n_cat, vrows, emb = tables.shape
    n_rows, n_num = num_feats.shape
    batch = nodes_v.shape[0]
    nodes = nodes_v.astype(jnp.int32)

    t_t = jnp.transpose(tables, (0, 2, 1))
    cat1d = jnp.transpose(cat_idx).reshape(-1).astype(jnp.int32)
    num1d = jnp.pad(jnp.transpose(num_feats).reshape(-1), (0, 96))

    w_t = jnp.transpose(W)
    b2d = jnp.broadcast_to(b[None, :], (8, 64))

    s1, s2 = 104, 264
    cidx1d = _sc_cidx(cat1d, nodes, batch, n_cat)

    tA = _tc_repack(t_t, 0, s1 // 8)
    emb_a = _sc_values(tA, cidx1d, batch, 0, s1)
    out = _tc_matmul(emb_a.reshape(s1, batch), w_t[:s1], b2d, batch)

    tB = _tc_repack(t_t, s1 // 8, (s2 - s1) // 8)
    emb_b = _sc_values(tB, cidx1d, batch, s1, s2,
                       num1d=num1d, nodes=nodes, n_num=n_num)
    w_b = jnp.concatenate([w_t[s1:s2], w_t[NLANE:]], axis=0)
    out = _tc_matmul(emb_b.reshape(s2 - s1 + n_num, batch), w_b, b2d, batch,
                     acc_in=out)

    tC = _tc_repack(t_t, s2 // 8, (NLANE - s2) // 8)
    emb_c = _sc_values(tC, cidx1d, batch, s2, NLANE)
    return _tc_matmul(emb_c.reshape(NLANE - s2, batch), w_t[s2:NLANE], b2d,
                      batch, acc_in=out)


# speedup vs baseline: 1.0387x; 1.0387x over previous
import functools

import jax
import jax.numpy as jnp
from jax import lax
from jax.experimental import pallas as pl
from jax.experimental.pallas import tpu as pltpu
from jax.experimental.pallas import tpu_sc as plsc

N_CAT = 26
EMB = 16
PADV = 100096
NROW = 100000
NLANE = N_CAT * EMB
SPLIT = 216
Q = 4096
NW = 32

_SC_PARAMS = dict(
    compiler_params=pltpu.CompilerParams(use_tc_tiling_on_sc=False,
                                         needs_layout_passes=False),
    mesh=plsc.VectorSubcoreMesh(core_axis_name="c", subcore_axis_name="s"),
)


def _sc_cidx(cat1d, nodes, batch, n_cat):
    nq = batch // Q

    @functools.partial(
        pl.kernel,
        out_type=jax.ShapeDtypeStruct((n_cat * batch,), jnp.int32),
        scratch_types=[
            pltpu.VMEM((PADV,), jnp.int32),
            pltpu.VMEM((Q,), jnp.int32),
            pltpu.VMEM((Q,), jnp.int32),
        ],
        **_SC_PARAMS,
    )
    def k(cat_hbm, nodes_hbm, cidx_hbm, big_vm, nodes_vm, vals_vm):
        wid = lax.axis_index("s") * 2 + lax.axis_index("c")

        @pl.when(wid < n_cat)
        def _():
            pltpu.sync_copy(cat_hbm.at[pl.ds(wid * NROW, NROW)],
                            big_vm.at[pl.ds(0, NROW)])
            for q in range(nq):
                pltpu.sync_copy(nodes_hbm.at[pl.ds(q * Q, Q)], nodes_vm)

                def g(i, carry):
                    for u in range(8):
                        i16 = nodes_vm[pl.ds(i * 128 + u * 16, 16)]
                        vals_vm[pl.ds(i * 128 + u * 16, 16)] = \
                            plsc.load_gather(big_vm, [i16])
                    return carry
                lax.fori_loop(0, Q // 128, g, 0)
                pltpu.sync_copy(vals_vm,
                                cidx_hbm.at[pl.ds(wid * batch + q * Q, Q)])

    return k(cat1d, nodes)


def _sc_values(t1d, cidx1d, batch, lane_lo, lane_hi, num1d=None, nodes=None,
               n_num=0):
    nq = batch // Q
    n_tab = lane_hi - lane_lo
    out_rows = n_tab + n_num
    extra = (num1d, nodes) if n_num else ()

    @functools.partial(
        pl.kernel,
        out_type=jax.ShapeDtypeStruct((out_rows * batch,), jnp.float32),
        scratch_types=[
            pltpu.VMEM((PADV,), jnp.float32),
            pltpu.VMEM((batch,), jnp.int32),
            pltpu.VMEM((Q,), jnp.float32),
        ],
        **_SC_PARAMS,
    )
    def k(t_hbm, cidx_hbm, *rest):
        (num_hbm, nodes_hbm) = rest[:2] if n_num else (None, None)
        out_hbm, big_vm, cidx_vm, vals_vm = rest[-4:]
        wid = lax.axis_index("s") * 2 + lax.axis_index("c")

        def quarters(out_row):
            for q in range(nq):
                def g(i, carry):
                    for u in range(8):
                        i16 = cidx_vm[pl.ds(q * Q + i * 128 + u * 16, 16)]
                        vals_vm[pl.ds(i * 128 + u * 16, 16)] = \
                            plsc.load_gather(big_vm, [i16])
                    return carry
                lax.fori_loop(0, Q // 128, g, 0)
                pltpu.sync_copy(
                    vals_vm,
                    out_hbm.at[pl.ds(out_row * batch + q * Q, Q)])

        def tjob(t, carry):
            j = wid + NW * t

            @pl.when(j < n_tab)
            def _():
                f = (lane_lo + j) // EMB
                pltpu.sync_copy(cidx_hbm.at[pl.ds(f * batch, batch)], cidx_vm)
                pltpu.sync_copy(t_hbm.at[pl.ds(j * PADV, PADV)], big_vm)
                quarters(j)
            return carry
        lax.fori_loop(0, (n_tab + NW - 1) // NW, tjob, 0)

        if n_num:
            kk = wid - (NW - n_num)

            @pl.when(kk >= 0)
            def _num_tail():
                pltpu.sync_copy(nodes_hbm, cidx_vm)
                pltpu.sync_copy(num_hbm.at[pl.ds(kk * NROW, PADV)], big_vm)
                quarters(n_tab + kk)

    return k(t1d, cidx1d, *extra)


def _tc_repack(t_t, blk_lo, n_blk):
    def body(in_ref, out_ref):
        for j in range(8):
            out_ref[pl.ds(j * PADV, PADV)] = in_ref[0, j, :]

    return pl.pallas_call(
        body,
        grid=(n_blk,),
        in_specs=[pl.BlockSpec(
            (1, 8, PADV), lambda b: ((b + blk_lo) // 2, (b + blk_lo) % 2, 0))],
        out_specs=pl.BlockSpec((8 * PADV,), lambda b: (b,)),
        out_shape=jax.ShapeDtypeStruct((n_blk * 8 * PADV,), jnp.float32),
    )(t_t)


def _tc_matmul(emb, w, b2d, batch, acc_in=None):
    blk = 2048
    kd = emb.shape[0]
    extra = (acc_in,) if acc_in is not None else ()

    def body(emb_ref, w_ref, b_ref, *rest):
        acc_ref = rest[0] if acc_in is not None else None
        out_ref = rest[-1]
        acc = jax.lax.dot_general(
            emb_ref[...], w_ref[...],
            dimension_numbers=(((0,), (0,)), ((), ())),
            preferred_element_type=jnp.float32,
            precision=lax.Precision.HIGHEST)
        acc += b_ref[0:1, :] if acc_in is None else acc_ref[...]
        out_ref[...] = acc

    return pl.pallas_call(
        body,
        grid=(batch // blk,),
        in_specs=[
            pl.BlockSpec((kd, blk), lambda i: (0, i)),
            pl.BlockSpec((kd, 64), lambda i: (0, 0)),
            pl.BlockSpec((8, 64), lambda i: (0, 0)),
        ] + ([pl.BlockSpec((blk, 64), lambda i: (i, 0))] if extra else []),
        out_specs=pl.BlockSpec((blk, 64), lambda i: (i, 0)),
        out_shape=jax.ShapeDtypeStruct((batch, 64), jnp.float32),
    )(emb, w, b2d, *extra)


def kernel(tables, num_feats, W, b, cat_idx, nodes_v):
    n_cat, vrows, emb = tables.shape
    n_rows, n_num = num_feats.shape
    batch = nodes_v.shape[0]
    nodes = nodes_v.astype(jnp.int32)

    t_t = jnp.transpose(tables, (0, 2, 1))
    cat1d = jnp.transpose(cat_idx).reshape(-1).astype(jnp.int32)
    num1d = jnp.pad(jnp.transpose(num_feats).reshape(-1), (0, 96))

    w_t = jnp.transpose(W)
    b2d = jnp.broadcast_to(b[None, :], (8, 64))

    nb1 = SPLIT // 8
    cidx1d = _sc_cidx(cat1d, nodes, batch, n_cat)

    tA = _tc_repack(t_t, 0, nb1)
    emb1 = _sc_values(tA, cidx1d, batch, 0, SPLIT)
    part1 = _tc_matmul(emb1.reshape(SPLIT, batch), w_t[:SPLIT], b2d, batch)

    tB = _tc_repack(t_t, nb1, NLANE // 8 - nb1)
    emb2 = _sc_values(tB, cidx1d, batch, SPLIT, NLANE,
                      num1d=num1d, nodes=nodes, n_num=n_num)
    return _tc_matmul(emb2.reshape(NLANE - SPLIT + n_num, batch),
                      w_t[SPLIT:], b2d, batch, acc_in=part1)


# speedup vs baseline: 1.1147x; 1.0731x over previous
import functools

import jax
import jax.numpy as jnp
from jax import lax
from jax.experimental import pallas as pl
from jax.experimental.pallas import tpu as pltpu
from jax.experimental.pallas import tpu_sc as plsc

N_CAT = 26
EMB = 16
PADV = 100096
NROW = 100000
NLANE = N_CAT * EMB
SPLIT = 216
Q = 4096
NW = 32

_SC_PARAMS = dict(
    compiler_params=pltpu.CompilerParams(use_tc_tiling_on_sc=False,
                                         needs_layout_passes=False),
    mesh=plsc.VectorSubcoreMesh(core_axis_name="c", subcore_axis_name="s"),
)


def _sc_cidx(cat1d, nodes, batch, n_cat):
    nq = batch // Q

    @functools.partial(
        pl.kernel,
        out_type=jax.ShapeDtypeStruct((n_cat * batch,), jnp.int32),
        scratch_types=[
            pltpu.VMEM((PADV,), jnp.int32),
            pltpu.VMEM((Q,), jnp.int32),
            pltpu.VMEM((Q,), jnp.int32),
        ],
        **_SC_PARAMS,
    )
    def k(cat_hbm, nodes_hbm, cidx_hbm, big_vm, nodes_vm, vals_vm):
        wid = lax.axis_index("s") * 2 + lax.axis_index("c")

        @pl.when(wid < n_cat)
        def _():
            pltpu.sync_copy(cat_hbm.at[pl.ds(wid * NROW, NROW)],
                            big_vm.at[pl.ds(0, NROW)])
            for q in range(nq):
                pltpu.sync_copy(nodes_hbm.at[pl.ds(q * Q, Q)], nodes_vm)

                def g(i, carry):
                    for u in range(8):
                        i16 = nodes_vm[pl.ds(i * 128 + u * 16, 16)]
                        vals_vm[pl.ds(i * 128 + u * 16, 16)] = \
                            plsc.load_gather(big_vm, [i16])
                    return carry
                lax.fori_loop(0, Q // 128, g, 0)
                pltpu.sync_copy(vals_vm,
                                cidx_hbm.at[pl.ds(wid * batch + q * Q, Q)])

    return k(cat1d, nodes)


def _sc_values(t1d, cidx1d, batch, lane_lo, lane_hi, num1d=None, nodes=None,
               n_num=0):
    nq = batch // Q
    n_tab = lane_hi - lane_lo
    out_rows = n_tab + n_num
    extra = (num1d, nodes) if n_num else ()

    @functools.partial(
        pl.kernel,
        out_type=jax.ShapeDtypeStruct((out_rows * batch,), jnp.float32),
        scratch_types=[
            pltpu.VMEM((PADV,), jnp.float32),
            pltpu.VMEM((batch,), jnp.int32),
            pltpu.VMEM((Q,), jnp.float32),
            pltpu.SemaphoreType.DMA,
            pltpu.SemaphoreType.DMA,
        ],
        **_SC_PARAMS,
    )
    def k(t_hbm, cidx_hbm, *rest):
        (num_hbm, nodes_hbm) = rest[:2] if n_num else (None, None)
        out_hbm, big_vm, cidx_vm, vals_vm, sem_a, sem_b = rest[-6:]
        wid = lax.axis_index("s") * 2 + lax.axis_index("c")

        def quarters(out_row):
            for q in range(nq):
                def g(i, carry):
                    for u in range(8):
                        i16 = cidx_vm[pl.ds(q * Q + i * 128 + u * 16, 16)]
                        vals_vm[pl.ds(i * 128 + u * 16, 16)] = \
                            plsc.load_gather(big_vm, [i16])
                    return carry
                lax.fori_loop(0, Q // 128, g, 0)
                pltpu.sync_copy(
                    vals_vm,
                    out_hbm.at[pl.ds(out_row * batch + q * Q, Q)])

        def tjob(t, carry):
            j = wid + NW * t

            @pl.when(j < n_tab)
            def _():
                f = (lane_lo + j) // EMB
                h1 = pltpu.async_copy(cidx_hbm.at[pl.ds(f * batch, batch)],
                                      cidx_vm, sem_a)
                h2 = pltpu.async_copy(t_hbm.at[pl.ds(j * PADV, PADV)],
                                      big_vm, sem_b)
                h1.wait()
                h2.wait()
                quarters(j)
            return carry
        lax.fori_loop(0, (n_tab + NW - 1) // NW, tjob, 0)

        if n_num:
            kk = wid - (NW - n_num)

            @pl.when(kk >= 0)
            def _num_tail():
                pltpu.sync_copy(nodes_hbm, cidx_vm)
                pltpu.sync_copy(num_hbm.at[pl.ds(kk * NROW, PADV)], big_vm)
                quarters(n_tab + kk)

    return k(t1d, cidx1d, *extra)


def _tc_repack(t_t, blk_lo, n_blk):
    def body(in_ref, out_ref):
        for j in range(8):
            out_ref[pl.ds(j * PADV, PADV)] = in_ref[0, j, :]

    return pl.pallas_call(
        body,
        grid=(n_blk,),
        in_specs=[pl.BlockSpec(
            (1, 8, PADV), lambda b: ((b + blk_lo) // 2, (b + blk_lo) % 2, 0))],
        out_specs=pl.BlockSpec((8 * PADV,), lambda b: (b,)),
        out_shape=jax.ShapeDtypeStruct((n_blk * 8 * PADV,), jnp.float32),
    )(t_t)


def _tc_matmul(emb1, emb2, w1, w2, b2d, batch):
    blk = 2048
    k1, k2 = emb1.shape[0], emb2.shape[0]

    def body(e1_ref, e2_ref, w1_ref, w2_ref, b_ref, out_ref):
        dn = (((0,), (0,)), ((), ()))
        acc = jax.lax.dot_general(
            w1_ref[...], e1_ref[...], dimension_numbers=dn,
            preferred_element_type=jnp.float32,
            precision=lax.Precision.HIGHEST)
        acc += jax.lax.dot_general(
            w2_ref[...], e2_ref[...], dimension_numbers=dn,
            preferred_element_type=jnp.float32,
            precision=lax.Precision.HIGHEST)
        out_ref[...] = acc + b_ref[:, 0:1]

    return pl.pallas_call(
        body,
        grid=(batch // blk,),
        in_specs=[
            pl.BlockSpec((k1, blk), lambda i: (0, i)),
            pl.BlockSpec((k2, blk), lambda i: (0, i)),
            pl.BlockSpec((k1, 64), lambda i: (0, 0)),
            pl.BlockSpec((k2, 64), lambda i: (0, 0)),
            pl.BlockSpec((64, 8), lambda i: (0, 0)),
        ],
        out_specs=pl.BlockSpec((64, blk), lambda i: (0, i)),
        out_shape=jax.ShapeDtypeStruct((64, batch), jnp.float32),
    )(emb1, emb2, w1, w2, b2d)


def kernel(tables, num_feats, W, b, cat_idx, nodes_v):
    n_cat, vrows, emb = tables.shape
    n_rows, n_num = num_feats.shape
    batch = nodes_v.shape[0]
    nodes = nodes_v.astype(jnp.int32)

    t_t = jnp.transpose(tables, (0, 2, 1))
    cat1d = jnp.transpose(cat_idx).reshape(-1).astype(jnp.int32)
    num1d = jnp.pad(jnp.transpose(num_feats).reshape(-1), (0, 96))

    w_t = jnp.transpose(W)

    nb1 = SPLIT // 8
    cidx1d = _sc_cidx(cat1d, nodes, batch, n_cat)

    tA = _tc_repack(t_t, 0, nb1)
    emb1 = _sc_values(tA, cidx1d, batch, 0, SPLIT)

    tB = _tc_repack(t_t, nb1, NLANE // 8 - nb1)
    emb2 = _sc_values(tB, cidx1d, batch, SPLIT, NLANE,
                      num1d=num1d, nodes=nodes, n_num=n_num)

    b2d = jnp.broadcast_to(b[:, None], (64, 8))
    out_t = _tc_matmul(emb1.reshape(SPLIT, batch),
                       emb2.reshape(NLANE - SPLIT + n_num, batch),
                       w_t[:SPLIT], w_t[SPLIT:], b2d, batch)
    return jnp.transpose(out_t)
